# baseline (speedup 1.0000x reference)
import functools

import jax
import jax.numpy as jnp
from jax import lax
from jax.experimental import pallas as pl
from jax.experimental.pallas import tpu as pltpu
from jax.experimental.pallas import tpu_sc as plsc

N = 10000
E = 320000
D = 128
H = 128
Z = 64
TAU = 0.8
NUM_CLUSTER = 20

NC = 2
NS = 16
NW = NC * NS
NP = 10240
RPT = NP // NS


def _seg_sum_multi_kernel(k, ch, g):
    per_w = E // NW
    n_ch = per_w // ch
    n_g = ch // g
    zr = min(ch - ch % 8, RPT)
    assert n_ch % 2 == 0
    mesh = plsc.VectorSubcoreMesh(core_axis_name="c", subcore_axis_name="s",
                                  num_cores=NC, num_subcores=NS)

    @functools.partial(
        pl.kernel,
        out_type=jax.ShapeDtypeStruct((k, NC, NP, 64), jnp.float32),
        mesh=mesh,
        compiler_params=pltpu.CompilerParams(use_tc_tiling_on_sc=False),
        scratch_types=[
            pltpu.VMEM((2, n_g, g), jnp.int32),
            pltpu.VMEM((2, n_g, g), jnp.int32),
            pltpu.VMEM((ch, 64), jnp.float32),
            pltpu.VMEM((ch, 64), jnp.float32),
            pltpu.VMEM_SHARED((NP, 64), jnp.float32),
            pltpu.SemaphoreType.DMA,
            pltpu.SemaphoreType.DMA,
        ],
    )
    def body(*refs):
        xs = refs[:k]
        ers = refs[k:2 * k]
        zero_hbm = refs[2 * k]
        out_hbm = refs[2 * k + 1]
        dst_v, src_v, rows_a, rows_b, acc_sh, sem_g, sem_s = refs[2 * k + 2:]
        cid = lax.axis_index("c")
        sid = lax.axis_index("s")
        wid = sid * NC + cid

        for p in range(k):
            x_hbm, er_hbm = xs[p], ers[p]
            pltpu.sync_copy(zero_hbm, rows_a.at[pl.ds(0, zr)])
            for o in range(0, RPT, zr):
                m = min(zr, RPT - o)
                pltpu.sync_copy(rows_a.at[pl.ds(0, m)],
                                acc_sh.at[pl.ds(sid * RPT + o, m)])
            plsc.subcore_barrier()

            def pair(t, carry, x_hbm=x_hbm, er_hbm=er_hbm):
                a = 2 * t
                b = a + 1
                pltpu.sync_copy(er_hbm.at[0, wid, a], dst_v.at[0])
                pltpu.sync_copy(er_hbm.at[1, wid, a], src_v.at[0])
                ga = [
                    pltpu.async_copy(x_hbm.at[src_v.at[0, i]],
                                     rows_a.at[pl.ds(i * g, g)], sem_g)
                    for i in range(n_g)
                ]
                pltpu.sync_copy(er_hbm.at[0, wid, b], dst_v.at[1])
                pltpu.sync_copy(er_hbm.at[1, wid, b], src_v.at[1])
                for de in ga:
                    de.wait()
                sa = [
                    pltpu.async_copy(rows_a.at[pl.ds(i * g, g)],
                                     acc_sh.at[dst_v.at[0, i]], sem_s,
                                     add=True)
                    for i in range(n_g)
                ]
                gb = [
                    pltpu.async_copy(x_hbm.at[src_v.at[1, i]],
                                     rows_b.at[pl.ds(i * g, g)], sem_g)
                    for i in range(n_g)
                ]
                for de in gb:
                    de.wait()
                sb = [
                    pltpu.async_copy(rows_b.at[pl.ds(i * g, g)],
                                     acc_sh.at[dst_v.at[1, i]], sem_s,
                                     add=True)
                    for i in range(n_g)
                ]
                for de in sa + sb:
                    de.wait()
                return carry

            lax.fori_loop(0, n_ch // 2, pair, 0)
            plsc.subcore_barrier()

            for o in range(0, RPT, zr):
                m = min(zr, RPT - o)
                pltpu.sync_copy(acc_sh.at[pl.ds(sid * RPT + o, m)],
                                rows_a.at[pl.ds(0, m)])
                pltpu.sync_copy(rows_a.at[pl.ds(0, m)],
                                out_hbm.at[p, cid, pl.ds(sid * RPT + o, m)])
            if p + 1 < k:
                plsc.subcore_barrier()

    return body


_SS_CH = 200
_SS_G = 100
_SS_NCH = (E // NW) // _SS_CH


@functools.lru_cache(maxsize=None)
def _get_seg_sum_multi(k):
    return _seg_sum_multi_kernel(k, _SS_CH, _SS_G)


def _seg_sum_batch(xs, ers):
    zeros = jnp.zeros((min(_SS_CH - _SS_CH % 8, RPT), 64), jnp.float32)
    out = _get_seg_sum_multi(len(xs))(*xs, *ers, zeros)
    return [out[p, 0, :N] + out[p, 1, :N] for p in range(len(xs))]


_CNT_CH = 2000
_CNT_NCH = (E // NW) // _CNT_CH


def _make_count_kernel(n_lists):
    mesh = plsc.VectorSubcoreMesh(core_axis_name="c", subcore_axis_name="s",
                                  num_cores=NC, num_subcores=NS)

    @functools.partial(
        pl.kernel,
        out_type=jax.ShapeDtypeStruct((n_lists, NW, 1, NP), jnp.float32),
        mesh=mesh,
        compiler_params=pltpu.CompilerParams(use_tc_tiling_on_sc=False,
                                             needs_layout_passes=False),
        scratch_types=[
            pltpu.VMEM((_CNT_CH,), jnp.int32),
            pltpu.VMEM((NP,), jnp.float32),
        ],
    )
    def body(*refs):
        ers = refs[:n_lists]
        out_hbm = refs[n_lists]
        didx, cnt_v = refs[n_lists + 1:]
        cid = lax.axis_index("c")
        sid = lax.axis_index("s")
        wid = sid * NC + cid
        ones = jnp.ones((16,), jnp.float32)
        zeros = jnp.zeros((16,), jnp.float32)

        for l in range(n_lists):
            def zero(v, carry):
                cnt_v[pl.ds(v * 16, 16)] = zeros
                return carry
            lax.fori_loop(0, NP // 16, zero, 0)

            def chunk(c, carry, er=ers[l]):
                pltpu.sync_copy(er.at[wid, c, 0], didx)

                def group(v, carry2):
                    idx = didx[pl.ds(v * 16, 16)]
                    plsc.addupdate_scatter(cnt_v, [idx], ones)
                    return carry2

                lax.fori_loop(0, _CNT_CH // 16, group, 0)
                return carry

            lax.fori_loop(0, _CNT_NCH, chunk, 0)
            pltpu.sync_copy(cnt_v, out_hbm.at[l, wid, 0])

    return body


_get_count7 = functools.lru_cache(maxsize=None)(lambda: _make_count_kernel(7))

_BM = 1024
_NPAD = 10240


def _contrast_body(z1_ref, z2_ref, row_ref, col_ref, diag_ref):
    i = pl.program_id(0)
    j = pl.program_id(1)
    s = lax.dot_general(z1_ref[...], z2_ref[...],
                        (((1,), (1,)), ((), ())),
                        preferred_element_type=jnp.float32) * (1.0 / TAU)
    e = jnp.exp(s)
    rs = jnp.sum(e, axis=1)
    cs = jnp.sum(e, axis=0)

    @pl.when(j == 0)
    def _():
        row_ref[0, pl.ds(i * _BM, _BM)] = rs

    @pl.when(j != 0)
    def _():
        row_ref[0, pl.ds(i * _BM, _BM)] += rs

    @pl.when(i == 0)
    def _():
        col_ref[0, pl.ds(j * _BM, _BM)] = cs

    @pl.when(i != 0)
    def _():
        col_ref[0, pl.ds(j * _BM, _BM)] += cs

    @pl.when(i == j)
    def _():
        diag_ref[0, pl.ds(i * _BM, _BM)] = (
            jnp.sum(z1_ref[...] * z2_ref[...], axis=1) * (1.0 / TAU))


def _contrast(z1, z2):
    pad = _NPAD - N
    z1p = jnp.pad(z1, ((0, pad), (0, 0)))
    z2p = jnp.pad(z2, ((0, pad), (0, 0)))
    grid = (_NPAD // _BM, _NPAD // _BM)
    row, col, diag = pl.pallas_call(
        _contrast_body,
        grid=grid,
        in_specs=[
            pl.BlockSpec((_BM, Z), lambda i, j: (i, 0)),
            pl.BlockSpec((_BM, Z), lambda i, j: (j, 0)),
        ],
        out_specs=[
            pl.BlockSpec((1, _NPAD), lambda i, j: (0, 0)),
            pl.BlockSpec((1, _NPAD), lambda i, j: (0, 0)),
            pl.BlockSpec((1, _NPAD), lambda i, j: (0, 0)),
        ],
        out_shape=[
            jax.ShapeDtypeStruct((1, _NPAD), jnp.float32),
            jax.ShapeDtypeStruct((1, _NPAD), jnp.float32),
            jax.ShapeDtypeStruct((1, _NPAD), jnp.float32),
        ],
    )(z1p, z2p)
    return row[0, :N] - pad, col[0, :N] - pad, diag[0, :N]


def _reshape_edges(edge, ch, g):
    per_w = E // NW
    return edge.reshape(2, NW, per_w // ch, ch // g, g)


def _l2norm(x):
    return x / (jnp.linalg.norm(x, axis=1, keepdims=True) + 1e-8)


def kernel(feats_0, feats_1, feats_2, mask_feat, fc_W0, fc_b0, fc_W1, fc_b1,
           fc_W2, fc_b2, agg_W0, agg_W1, gcn_W1, gcn_b1, gcn_W2, gcn_b2,
           proj_W, proj_b, att_W, att_b, att_a, nei_edge_0, nei_edge_1,
           mask_edge_0, mask_edge_1, norm_edge_0, norm_edge_1, adj_edge,
           num_cluster):
    edges = [nei_edge_0, nei_edge_1, mask_edge_0, mask_edge_1, norm_edge_0,
             norm_edge_1, adj_edge]
    cnt_ers = [e[0].reshape(NW, _CNT_NCH, 1, _CNT_CH) for e in edges]
    cparts = _get_count7()(*cnt_ers)
    cnt = cparts[:, :, 0, :N].sum(axis=1)
    recip = 1.0 / jnp.maximum(cnt, 1.0)
    r_nei0, r_nei1, r_mask0, r_mask1, r_norm0, r_norm1, r_adj = [
        recip[i] for i in range(7)]

    er_nei = [_reshape_edges(e, _SS_CH, _SS_G) for e in (nei_edge_0,
                                                         nei_edge_1)]
    er64 = {k: _reshape_edges(e, _SS_CH, _SS_G) for k, e in
            zip(("mask0", "mask1", "norm0", "norm1", "adj"),
                (mask_edge_0, mask_edge_1, norm_edge_0, norm_edge_1,
                 adj_edge))}

    elu = jax.nn.elu
    h_tar = elu(feats_0 @ fc_W0 + fc_b0)
    h_mask = elu(mask_feat @ fc_W0 + fc_b0)
    h_nei = [elu(feats_1 @ fc_W1 + fc_b1), elu(feats_2 @ fc_W2 + fc_b2)]

    parts = _seg_sum_batch(
        [h_nei[0][:, :64], h_nei[0][:, 64:], h_nei[1][:, :64],
         h_nei[1][:, 64:]],
        [er_nei[0], er_nei[0], er_nei[1], er_nei[1]])
    t = []
    for i, (r, agg_W) in enumerate(((r_nei0, agg_W0), (r_nei1, agg_W1))):
        h_agg = jnp.concatenate(parts[2 * i:2 * i + 2], axis=1) * r[:, None]
        t.append(h_agg @ agg_W)

    hs = [elu(h_tar + t[0]), elu(h_mask + t[0]), elu(h_tar + t[1]),
          elu(h_mask + t[1]), h_tar]
    ers5 = [er64["norm0"], er64["mask0"], er64["norm1"], er64["mask1"],
            er64["adj"]]
    rs5 = [r_norm0, r_mask0, r_norm1, r_mask1, r_adj]
    ps = [h @ gcn_W1 + gcn_b1 for h in hs]
    m1 = _seg_sum_batch(ps, ers5)
    qs = [jax.nn.relu(m * r[:, None]) @ gcn_W2 + gcn_b2
          for m, r in zip(m1, rs5)]

    m2 = _seg_sum_batch(qs, ers5)
    outs = [m * r[:, None] for m, r in zip(m2, rs5)]
    z_new = outs[:4]
    z_coarse = outs[4]

    z_coarse = _l2norm(jnp.tanh(z_coarse @ proj_W + proj_b))

    z_new = [_l2norm(zt) for zt in z_new]
    zs = jnp.stack(z_new)
    w = (jnp.tanh(zs @ att_W + att_b) @ att_a).mean(axis=1)
    beta = jax.nn.softmax(w)
    z = jnp.einsum('v,vnd->nd', beta, zs)
    z_pro = _l2norm(jnp.tanh(z @ proj_W + proj_b))

    row_se, col_se, diag = _contrast(z_coarse, z_pro)
    l1 = (jnp.log(row_se) - diag).mean()
    l2_ = (jnp.log(col_se) - diag).mean()
    loss_info = 0.5 * (l1 + l2_)

    assign = jnp.arange(N) % NUM_CLUSTER
    protos = _l2norm(z_pro.reshape(N // NUM_CLUSTER, NUM_CLUSTER, Z).mean(0))
    logits = z_pro @ protos.T / TAU
    pos = jnp.take_along_axis(logits, assign[:, None], axis=1)[:, 0]
    loss_proto = (jax.nn.logsumexp(logits, axis=1) - pos).mean()
    return loss_info + loss_proto


# speedup vs baseline: 1.2238x; 1.2238x over previous
import functools

import jax
import jax.numpy as jnp
from jax import lax
from jax.experimental import pallas as pl
from jax.experimental.pallas import tpu as pltpu
from jax.experimental.pallas import tpu_sc as plsc

N = 10000
E = 320000
D = 128
H = 128
Z = 64
TAU = 0.8
NUM_CLUSTER = 20

NC = 2
NS = 16
NW = NC * NS
NP = 10240
RPT = NP // NS


def _seg_sum_multi_kernel(k, ch, g):
    per_w = E // NW
    n_ch = per_w // ch
    n_g = ch // g
    zr = min(ch - ch % 8, RPT)
    assert n_ch % 2 == 0
    mesh = plsc.VectorSubcoreMesh(core_axis_name="c", subcore_axis_name="s",
                                  num_cores=NC, num_subcores=NS)

    @functools.partial(
        pl.kernel,
        out_type=jax.ShapeDtypeStruct((k, NC, NP, 64), jnp.float32),
        mesh=mesh,
        compiler_params=pltpu.CompilerParams(use_tc_tiling_on_sc=False),
        scratch_types=[
            pltpu.VMEM((2, n_g, g), jnp.int32),
            pltpu.VMEM((2, n_g, g), jnp.int32),
            pltpu.VMEM((ch, 64), jnp.float32),
            pltpu.VMEM((ch, 64), jnp.float32),
            pltpu.VMEM_SHARED((NP, 64), jnp.float32),
            pltpu.SemaphoreType.DMA,
            pltpu.SemaphoreType.DMA,
        ],
    )
    def body(*refs):
        xs = refs[:k]
        ers = refs[k:2 * k]
        zero_hbm = refs[2 * k]
        out_hbm = refs[2 * k + 1]
        dst_v, src_v, rows_a, rows_b, acc_sh, sem_g, sem_s = refs[2 * k + 2:]
        cid = lax.axis_index("c")
        sid = lax.axis_index("s")
        wid = sid * NC + cid

        for p in range(k):
            x_hbm, er_hbm = xs[p], ers[p]
            pltpu.sync_copy(zero_hbm, rows_a.at[pl.ds(0, zr)])
            for o in range(0, RPT, zr):
                m = min(zr, RPT - o)
                pltpu.sync_copy(rows_a.at[pl.ds(0, m)],
                                acc_sh.at[pl.ds(sid * RPT + o, m)])
            plsc.subcore_barrier()

            def pair(t, carry, x_hbm=x_hbm, er_hbm=er_hbm):
                a = 2 * t
                b = a + 1
                pltpu.sync_copy(er_hbm.at[0, wid, a], dst_v.at[0])
                pltpu.sync_copy(er_hbm.at[1, wid, a], src_v.at[0])
                ga = [
                    pltpu.async_copy(x_hbm.at[src_v.at[0, i]],
                                     rows_a.at[pl.ds(i * g, g)], sem_g)
                    for i in range(n_g)
                ]
                pltpu.sync_copy(er_hbm.at[0, wid, b], dst_v.at[1])
                pltpu.sync_copy(er_hbm.at[1, wid, b], src_v.at[1])
                for de in ga:
                    de.wait()
                sa = [
                    pltpu.async_copy(rows_a.at[pl.ds(i * g, g)],
                                     acc_sh.at[dst_v.at[0, i]], sem_s,
                                     add=True)
                    for i in range(n_g)
                ]
                gb = [
                    pltpu.async_copy(x_hbm.at[src_v.at[1, i]],
                                     rows_b.at[pl.ds(i * g, g)], sem_g)
                    for i in range(n_g)
                ]
                for de in gb:
                    de.wait()
                sb = [
                    pltpu.async_copy(rows_b.at[pl.ds(i * g, g)],
                                     acc_sh.at[dst_v.at[1, i]], sem_s,
                                     add=True)
                    for i in range(n_g)
                ]
                for de in sa + sb:
                    de.wait()
                return carry

            lax.fori_loop(0, n_ch // 2, pair, 0)
            plsc.subcore_barrier()

            for o in range(0, RPT, zr):
                m = min(zr, RPT - o)
                pltpu.sync_copy(acc_sh.at[pl.ds(sid * RPT + o, m)],
                                rows_a.at[pl.ds(0, m)])
                pltpu.sync_copy(rows_a.at[pl.ds(0, m)],
                                out_hbm.at[p, cid, pl.ds(sid * RPT + o, m)])
            if p + 1 < k:
                plsc.subcore_barrier()

    return body


_SS_CH = 500
_SS_G = 100
_SS_NCH = (E // NW) // _SS_CH


@functools.lru_cache(maxsize=None)
def _get_seg_sum_multi(k):
    return _seg_sum_multi_kernel(k, _SS_CH, _SS_G)


def _seg_sum_batch(xs, ers):
    zeros = jnp.zeros((min(_SS_CH - _SS_CH % 8, RPT), 64), jnp.float32)
    out = _get_seg_sum_multi(len(xs))(*xs, *ers, zeros)
    return [out[p, 0, :N] + out[p, 1, :N] for p in range(len(xs))]


_CNT_CH = 2000
_CNT_NCH = (E // NW) // _CNT_CH


def _make_count_kernel(n_lists):
    mesh = plsc.VectorSubcoreMesh(core_axis_name="c", subcore_axis_name="s",
                                  num_cores=NC, num_subcores=NS)

    @functools.partial(
        pl.kernel,
        out_type=jax.ShapeDtypeStruct((n_lists, NW, 1, NP), jnp.float32),
        mesh=mesh,
        compiler_params=pltpu.CompilerParams(use_tc_tiling_on_sc=False,
                                             needs_layout_passes=False),
        scratch_types=[
            pltpu.VMEM((_CNT_CH,), jnp.int32),
            pltpu.VMEM((NP,), jnp.float32),
        ],
    )
    def body(*refs):
        ers = refs[:n_lists]
        out_hbm = refs[n_lists]
        didx, cnt_v = refs[n_lists + 1:]
        cid = lax.axis_index("c")
        sid = lax.axis_index("s")
        wid = sid * NC + cid
        ones = jnp.ones((16,), jnp.float32)
        zeros = jnp.zeros((16,), jnp.float32)

        for l in range(n_lists):
            def zero(v, carry):
                cnt_v[pl.ds(v * 16, 16)] = zeros
                return carry
            lax.fori_loop(0, NP // 16, zero, 0)

            def chunk(c, carry, er=ers[l]):
                pltpu.sync_copy(er.at[wid, c, 0], didx)

                def group(v, carry2):
                    idx = didx[pl.ds(v * 16, 16)]
                    plsc.addupdate_scatter(cnt_v, [idx], ones)
                    return carry2

                lax.fori_loop(0, _CNT_CH // 16, group, 0)
                return carry

            lax.fori_loop(0, _CNT_NCH, chunk, 0)
            pltpu.sync_copy(cnt_v, out_hbm.at[l, wid, 0])

    return body


_get_count7 = functools.lru_cache(maxsize=None)(lambda: _make_count_kernel(7))

_BM = 1024
_NPAD = 10240


def _contrast_body(z1_ref, z2_ref, row_ref, col_ref, diag_ref):
    i = pl.program_id(0)
    j = pl.program_id(1)
    s = lax.dot_general(z1_ref[...], z2_ref[...],
                        (((1,), (1,)), ((), ())),
                        preferred_element_type=jnp.float32) * (1.0 / TAU)
    e = jnp.exp(s)
    rs = jnp.sum(e, axis=1)
    cs = jnp.sum(e, axis=0)

    @pl.when(j == 0)
    def _():
        row_ref[0, pl.ds(i * _BM, _BM)] = rs

    @pl.when(j != 0)
    def _():
        row_ref[0, pl.ds(i * _BM, _BM)] += rs

    @pl.when(i == 0)
    def _():
        col_ref[0, pl.ds(j * _BM, _BM)] = cs

    @pl.when(i != 0)
    def _():
        col_ref[0, pl.ds(j * _BM, _BM)] += cs

    @pl.when(i == j)
    def _():
        diag_ref[0, pl.ds(i * _BM, _BM)] = (
            jnp.sum(z1_ref[...] * z2_ref[...], axis=1) * (1.0 / TAU))


def _contrast(z1, z2):
    pad = _NPAD - N
    z1p = jnp.pad(z1, ((0, pad), (0, 0)))
    z2p = jnp.pad(z2, ((0, pad), (0, 0)))
    grid = (_NPAD // _BM, _NPAD // _BM)
    row, col, diag = pl.pallas_call(
        _contrast_body,
        grid=grid,
        in_specs=[
            pl.BlockSpec((_BM, Z), lambda i, j: (i, 0)),
            pl.BlockSpec((_BM, Z), lambda i, j: (j, 0)),
        ],
        out_specs=[
            pl.BlockSpec((1, _NPAD), lambda i, j: (0, 0)),
            pl.BlockSpec((1, _NPAD), lambda i, j: (0, 0)),
            pl.BlockSpec((1, _NPAD), lambda i, j: (0, 0)),
        ],
        out_shape=[
            jax.ShapeDtypeStruct((1, _NPAD), jnp.float32),
            jax.ShapeDtypeStruct((1, _NPAD), jnp.float32),
            jax.ShapeDtypeStruct((1, _NPAD), jnp.float32),
        ],
    )(z1p, z2p)
    return row[0, :N] - pad, col[0, :N] - pad, diag[0, :N]


def _reshape_edges(edge, ch, g):
    per_w = E // NW
    return edge.reshape(2, NW, per_w // ch, ch // g, g)


def _l2norm(x):
    return x / (jnp.linalg.norm(x, axis=1, keepdims=True) + 1e-8)


def kernel(feats_0, feats_1, feats_2, mask_feat, fc_W0, fc_b0, fc_W1, fc_b1,
           fc_W2, fc_b2, agg_W0, agg_W1, gcn_W1, gcn_b1, gcn_W2, gcn_b2,
           proj_W, proj_b, att_W, att_b, att_a, nei_edge_0, nei_edge_1,
           mask_edge_0, mask_edge_1, norm_edge_0, norm_edge_1, adj_edge,
           num_cluster):
    edges = [nei_edge_0, nei_edge_1, mask_edge_0, mask_edge_1, norm_edge_0,
             norm_edge_1, adj_edge]
    cnt_ers = [e[0].reshape(NW, _CNT_NCH, 1, _CNT_CH) for e in edges]
    cparts = _get_count7()(*cnt_ers)
    cnt = cparts[:, :, 0, :N].sum(axis=1)
    recip = 1.0 / jnp.maximum(cnt, 1.0)
    r_nei0, r_nei1, r_mask0, r_mask1, r_norm0, r_norm1, r_adj = [
        recip[i] for i in range(7)]

    er_nei = [_reshape_edges(e, _SS_CH, _SS_G) for e in (nei_edge_0,
                                                         nei_edge_1)]
    er64 = {k: _reshape_edges(e, _SS_CH, _SS_G) for k, e in
            zip(("mask0", "mask1", "norm0", "norm1", "adj"),
                (mask_edge_0, mask_edge_1, norm_edge_0, norm_edge_1,
                 adj_edge))}

    elu = jax.nn.elu
    h_tar = elu(feats_0 @ fc_W0 + fc_b0)
    h_mask = elu(mask_feat @ fc_W0 + fc_b0)
    h_nei = [elu(feats_1 @ fc_W1 + fc_b1), elu(feats_2 @ fc_W2 + fc_b2)]

    parts = _seg_sum_batch(
        [h_nei[0][:, :64], h_nei[0][:, 64:], h_nei[1][:, :64],
         h_nei[1][:, 64:]],
        [er_nei[0], er_nei[0], er_nei[1], er_nei[1]])
    t = []
    for i, (r, agg_W) in enumerate(((r_nei0, agg_W0), (r_nei1, agg_W1))):
        h_agg = jnp.concatenate(parts[2 * i:2 * i + 2], axis=1) * r[:, None]
        t.append(h_agg @ agg_W)

    hs = [elu(h_tar + t[0]), elu(h_mask + t[0]), elu(h_tar + t[1]),
          elu(h_mask + t[1]), h_tar]
    ers5 = [er64["norm0"], er64["mask0"], er64["norm1"], er64["mask1"],
            er64["adj"]]
    rs5 = [r_norm0, r_mask0, r_norm1, r_mask1, r_adj]
    ps = [h @ gcn_W1 + gcn_b1 for h in hs]
    m1 = _seg_sum_batch(ps, ers5)
    qs = [jax.nn.relu(m * r[:, None]) @ gcn_W2 + gcn_b2
          for m, r in zip(m1, rs5)]

    m2 = _seg_sum_batch(qs, ers5)
    outs = [m * r[:, None] for m, r in zip(m2, rs5)]
    z_new = outs[:4]
    z_coarse = outs[4]

    z_coarse = _l2norm(jnp.tanh(z_coarse @ proj_W + proj_b))

    z_new = [_l2norm(zt) for zt in z_new]
    zs = jnp.stack(z_new)
    w = (jnp.tanh(zs @ att_W + att_b) @ att_a).mean(axis=1)
    beta = jax.nn.softmax(w)
    z = jnp.einsum('v,vnd->nd', beta, zs)
    z_pro = _l2norm(jnp.tanh(z @ proj_W + proj_b))

    row_se, col_se, diag = _contrast(z_coarse, z_pro)
    l1 = (jnp.log(row_se) - diag).mean()
    l2_ = (jnp.log(col_se) - diag).mean()
    loss_info = 0.5 * (l1 + l2_)

    assign = jnp.arange(N) % NUM_CLUSTER
    protos = _l2norm(z_pro.reshape(N // NUM_CLUSTER, NUM_CLUSTER, Z).mean(0))
    logits = z_pro @ protos.T / TAU
    pos = jnp.take_along_axis(logits, assign[:, None], axis=1)[:, 0]
    loss_proto = (jax.nn.logsumexp(logits, axis=1) - pos).mean()
    return loss_info + loss_proto
